# Initial kernel scaffold; baseline (speedup 1.0000x reference)
import jax
import jax.numpy as jnp
from jax.experimental import pallas as pl


def kernel(input_ids, token_type_ids, pos_embeds, pos_embeds_shape, atom_props, bond_props, mol_desc, emb_table, type_table, in_ring_table, charge_table, hybrid_table, chirality_table, aromatic_table, conjugated_table, stereo_table, ln_gamma, ln_beta):
    raise NotImplementedError("write your pallas kernel here")


# speedup vs baseline: 5.0366x; 5.0366x over previous
import functools

import jax
import jax.numpy as jnp
from jax import lax
from jax.experimental import pallas as pl
from jax.experimental.pallas import tpu as pltpu
from jax.experimental.pallas import tpu_sc as plsc

EPS = 1e-12


@functools.partial(jax.jit, static_argnames=("chunk",))
def _sc_gather(table, idx, chunk=128):
    V, D = table.shape
    N = idx.shape[0]
    info = plsc.get_sparse_core_info()
    NC, NS = info.num_cores, info.num_subcores
    NW = NC * NS
    assert N % (NW * chunk) == 0
    per_w = N // NW
    n_chunks = per_w // chunk
    mesh = plsc.VectorSubcoreMesh(core_axis_name="c", subcore_axis_name="s")

    @functools.partial(
        pl.kernel,
        mesh=mesh,
        out_type=jax.ShapeDtypeStruct((N, D), jnp.float32),
        scratch_types=[
            pltpu.VMEM((chunk,), jnp.int32),
            pltpu.VMEM((chunk, D), jnp.float32),
            pltpu.SemaphoreType.DMA,
        ],
    )
    def k(table_hbm, idx_hbm, out_hbm, idx_v, rows_v, sem):
        wid = lax.axis_index("s") * NC + lax.axis_index("c")
        base = wid * per_w

        def body(i, carry):
            off = base + i * chunk
            pltpu.sync_copy(idx_hbm.at[pl.ds(off, chunk)], idx_v)
            pltpu.async_copy(table_hbm.at[idx_v], rows_v, sem).wait()
            pltpu.sync_copy(rows_v, out_hbm.at[pl.ds(off, chunk)])
            return carry

        lax.fori_loop(0, n_chunks, body, 0)

    return k(table, idx)


def _masked_lookup(table_ref, idx, n):
    acc = jnp.where(idx == 0, 1.0, 0.0) * table_ref[0:1, :]
    for kk in range(1, n):
        acc += jnp.where(idx == kk, 1.0, 0.0) * table_ref[kk:kk + 1, :]
    return acc


def _fuse_body(D, g_ref, pos_ref, meta_ref, md_ref, type_ref, ring_ref,
               charge_ref, hyb_ref, chi_ref, aro_ref, conj_ref, ste_ref,
               gam_ref, bet_ref, out_ref):
    meta = meta_ref[...]
    tt = meta[:, 0:1]
    emb = g_ref[...][:, :D]
    md = md_ref[...]
    emb = emb * (1.0 + jnp.where(tt == 3, jnp.tanh(md), 0.0))

    type_emb = _masked_lookup(type_ref, tt, 4)

    ape = jnp.concatenate([
        _masked_lookup(ring_ref, meta[:, 1:2], 3),
        _masked_lookup(charge_ref, meta[:, 2:3], 4),
        _masked_lookup(hyb_ref, meta[:, 3:4], 9),
        _masked_lookup(chi_ref, meta[:, 4:5], 5),
    ], axis=1)
    bpe = jnp.concatenate([
        _masked_lookup(aro_ref, meta[:, 5:6], 3),
        _masked_lookup(conj_ref, meta[:, 6:7], 3),
        _masked_lookup(ste_ref, meta[:, 7:8], 7),
    ], axis=1)
    prop = (jnp.where(tt == 1, 1.0, 0.0) * ape
            + jnp.where(tt == 2, 1.0, 0.0) * bpe)

    full = jnp.concatenate([emb, pos_ref[...], type_emb, prop], axis=1)
    hid = full.shape[1]
    mean = jnp.sum(full, axis=1, keepdims=True) * (1.0 / hid)
    cen = full - mean
    var = jnp.sum(cen * cen, axis=1, keepdims=True) * (1.0 / hid)
    inv = lax.rsqrt(var + EPS)
    out_ref[...] = cen * inv * gam_ref[...] + bet_ref[...]


@functools.partial(jax.jit, static_argnames=("block",))
def _tc_fuse(gathered, pos, meta, md, type_table, ring, charge, hyb, chi,
             aro, conj, ste, gamma, beta, block=256):
    N, Dp = gathered.shape
    D = type_table.shape[1]
    posd = pos.shape[1]
    hid = D + posd + D + D
    assert N % block == 0
    grid = (N // block,)

    def tok_spec(d):
        return pl.BlockSpec((block, d), lambda i: (i, 0))

    def full_spec(shape):
        return pl.BlockSpec(shape, lambda i: (0, 0))

    return pl.pallas_call(
        functools.partial(_fuse_body, D),
        grid=grid,
        in_specs=[
            tok_spec(Dp), tok_spec(posd), tok_spec(meta.shape[1]), tok_spec(1),
            full_spec(type_table.shape), full_spec(ring.shape),
            full_spec(charge.shape), full_spec(hyb.shape),
            full_spec(chi.shape), full_spec(aro.shape),
            full_spec(conj.shape), full_spec(ste.shape),
            full_spec((1, hid)), full_spec((1, hid)),
        ],
        out_specs=pl.BlockSpec((block, hid), lambda i: (i, 0)),
        out_shape=jax.ShapeDtypeStruct((N, hid), jnp.float32),
    )(gathered, pos, meta, md, type_table, ring, charge, hyb, chi,
      aro, conj, ste, gamma.reshape(1, hid), beta.reshape(1, hid))


def kernel(input_ids, token_type_ids, pos_embeds, pos_embeds_shape,
           atom_props, bond_props, mol_desc, emb_table, type_table,
           in_ring_table, charge_table, hybrid_table, chirality_table,
           aromatic_table, conjugated_table, stereo_table, ln_gamma, ln_beta):
    B, L = input_ids.shape
    N = B * L
    posd = pos_embeds.shape[1] // L
    D = emb_table.shape[1]
    hid = ln_gamma.shape[0]

    ids = input_ids.reshape(N).astype(jnp.int32)
    Dp = ((D + 127) // 128) * 128
    table_p = jnp.pad(emb_table, ((0, 0), (0, Dp - D)))
    gathered = _sc_gather(table_p, ids)

    meta = jnp.concatenate([
        token_type_ids.reshape(N, 1),
        atom_props.reshape(N, 4),
        bond_props.reshape(N, 3),
    ], axis=1).astype(jnp.int32)
    pos = pos_embeds.reshape(N, posd)
    md = mol_desc.reshape(N, 1)

    out = _tc_fuse(gathered, pos, meta, md, type_table, in_ring_table,
                   charge_table, hybrid_table, chirality_table,
                   aromatic_table, conjugated_table, stereo_table,
                   ln_gamma, ln_beta)
    return out.reshape(B, L, hid)


# speedup vs baseline: 5.7726x; 1.1461x over previous
import functools

import jax
import jax.numpy as jnp
from jax import lax
from jax.experimental import pallas as pl
from jax.experimental.pallas import tpu as pltpu
from jax.experimental.pallas import tpu_sc as plsc

EPS = 1e-12


@functools.partial(jax.jit, static_argnames=("chunk",))
def _sc_gather(table, idx, chunk=128):
    V, D = table.shape
    N = idx.shape[0]
    info = plsc.get_sparse_core_info()
    NC, NS = info.num_cores, info.num_subcores
    NW = NC * NS
    assert N % (NW * chunk) == 0
    per_w = N // NW
    n_chunks = per_w // chunk
    mesh = plsc.VectorSubcoreMesh(core_axis_name="c", subcore_axis_name="s")

    @functools.partial(
        pl.kernel,
        mesh=mesh,
        out_type=jax.ShapeDtypeStruct((N, D), jnp.float32),
        scratch_types=[
            pltpu.VMEM((chunk,), jnp.int32),
            pltpu.VMEM((chunk, D), jnp.float32),
            pltpu.SemaphoreType.DMA,
        ],
    )
    def k(table_hbm, idx_hbm, out_hbm, idx_v, rows_v, sem):
        wid = lax.axis_index("s") * NC + lax.axis_index("c")
        base = wid * per_w

        def body(i, carry):
            off = base + i * chunk
            pltpu.sync_copy(idx_hbm.at[pl.ds(off, chunk)], idx_v)
            pltpu.async_copy(table_hbm.at[idx_v], rows_v, sem).wait()
            pltpu.sync_copy(rows_v, out_hbm.at[pl.ds(off, chunk)])
            return carry

        lax.fori_loop(0, n_chunks, body, 0)

    return k(table, idx)


def _fuse_body(D, posd, offs, g_ref, pos_ref, meta_ref, md_ref, w_ref,
               gam_ref, bet_ref, out_ref):
    meta = meta_ref[...]
    tt = meta[:, 0:1]
    emb = g_ref[...][:, :D]
    emb = emb * (1.0 + jnp.where(tt == 3, jnp.tanh(md_ref[...]), 0.0))

    j = lax.broadcasted_iota(jnp.int32, (1, w_ref.shape[0]), 1)
    is_atom = tt == 1
    is_bond = tt == 2
    oh = j == (tt + offs[0])
    oh |= is_atom & ((j == meta[:, 1:2] + offs[1])
                     | (j == meta[:, 2:3] + offs[2])
                     | (j == meta[:, 3:4] + offs[3])
                     | (j == meta[:, 4:5] + offs[4]))
    oh |= is_bond & ((j == meta[:, 5:6] + offs[5])
                     | (j == meta[:, 6:7] + offs[6])
                     | (j == meta[:, 7:8] + offs[7]))
    contrib = jnp.dot(oh.astype(jnp.float32), w_ref[...],
                      preferred_element_type=jnp.float32)

    T = emb.shape[0]
    hid = out_ref.shape[1]
    base = jnp.concatenate(
        [emb, pos_ref[...], jnp.zeros((T, hid - D - posd), jnp.float32)],
        axis=1)
    full = base + contrib
    mean = jnp.sum(full, axis=1, keepdims=True) * (1.0 / hid)
    cen = full - mean
    var = jnp.sum(cen * cen, axis=1, keepdims=True) * (1.0 / hid)
    inv = lax.rsqrt(var + EPS)
    out_ref[...] = cen * inv * gam_ref[...] + bet_ref[...]


@functools.partial(jax.jit, static_argnames=("D", "posd", "offs", "block"))
def _tc_fuse(gathered, pos, meta, md, w, gamma, beta, D, posd, offs,
             block=256):
    N, Dp = gathered.shape
    hid = w.shape[1]
    assert N % block == 0
    grid = (N // block,)

    def tok_spec(d):
        return pl.BlockSpec((block, d), lambda i: (i, 0))

    def full_spec(shape):
        return pl.BlockSpec(shape, lambda i: (0, 0))

    return pl.pallas_call(
        functools.partial(_fuse_body, D, posd, offs),
        grid=grid,
        in_specs=[
            tok_spec(Dp), tok_spec(posd), tok_spec(meta.shape[1]), tok_spec(1),
            full_spec(w.shape), full_spec((1, hid)), full_spec((1, hid)),
        ],
        out_specs=pl.BlockSpec((block, hid), lambda i: (i, 0)),
        out_shape=jax.ShapeDtypeStruct((N, hid), jnp.float32),
    )(gathered, pos, meta, md, w, gamma.reshape(1, hid), beta.reshape(1, hid))


def kernel(input_ids, token_type_ids, pos_embeds, pos_embeds_shape,
           atom_props, bond_props, mol_desc, emb_table, type_table,
           in_ring_table, charge_table, hybrid_table, chirality_table,
           aromatic_table, conjugated_table, stereo_table, ln_gamma, ln_beta):
    B, L = input_ids.shape
    N = B * L
    posd = pos_embeds.shape[1] // L
    D = emb_table.shape[1]
    hid = ln_gamma.shape[0]

    ids = input_ids.reshape(N).astype(jnp.int32)
    Dp = ((D + 127) // 128) * 128
    table_p = jnp.pad(emb_table, ((0, 0), (0, Dp - D)))
    gathered = _sc_gather(table_p, ids)

    meta = jnp.concatenate([
        token_type_ids.reshape(N, 1),
        atom_props.reshape(N, 4),
        bond_props.reshape(N, 3),
    ], axis=1).astype(jnp.int32)
    pos = pos_embeds.reshape(N, posd)
    md = mol_desc.reshape(N, 1)

    t0 = D + posd
    p0 = t0 + D
    per4 = in_ring_table.shape[1]
    per3 = aromatic_table.shape[1]
    tables = [
        (type_table, t0),
        (in_ring_table, p0),
        (charge_table, p0 + per4),
        (hybrid_table, p0 + 2 * per4),
        (chirality_table, p0 + 3 * per4),
        (aromatic_table, p0),
        (conjugated_table, p0 + per3),
        (stereo_table, p0 + 2 * per3),
    ]
    w = jnp.zeros((128, hid), jnp.float32)
    offs = []
    r = 0
    for tab, col in tables:
        n, d = tab.shape
        w = w.at[r:r + n, col:col + d].set(tab)
        offs.append(r)
        r += n

    out = _tc_fuse(gathered, pos, meta, md, w, ln_gamma, ln_beta,
                   D, posd, tuple(offs))
    return out.reshape(B, L, hid)


# speedup vs baseline: 6.5200x; 1.1295x over previous
import functools

import jax
import jax.numpy as jnp
from jax import lax
from jax.experimental import pallas as pl
from jax.experimental.pallas import tpu as pltpu
from jax.experimental.pallas import tpu_sc as plsc

EPS = 1e-12


@functools.partial(jax.jit, static_argnames=("chunk",))
def _sc_gather(table, idx, chunk=128):
    V, D = table.shape
    N = idx.shape[0]
    info = plsc.get_sparse_core_info()
    NC, NS = info.num_cores, info.num_subcores
    NW = NC * NS
    assert N % (NW * chunk) == 0
    per_w = N // NW
    n_chunks = per_w // chunk
    mesh = plsc.VectorSubcoreMesh(core_axis_name="c", subcore_axis_name="s")

    @functools.partial(
        pl.kernel,
        mesh=mesh,
        out_type=jax.ShapeDtypeStruct((N, D), jnp.float32),
        scratch_types=[
            pltpu.VMEM((chunk,), jnp.int32),
            pltpu.VMEM((chunk, D), jnp.float32),
            pltpu.SemaphoreType.DMA,
        ],
    )
    def k(table_hbm, idx_hbm, out_hbm, idx_v, rows_v, sem):
        wid = lax.axis_index("s") * NC + lax.axis_index("c")
        base = wid * per_w

        def body(i, carry):
            off = base + i * chunk
            pltpu.sync_copy(idx_hbm.at[pl.ds(off, chunk)], idx_v)
            pltpu.async_copy(table_hbm.at[idx_v], rows_v, sem).wait()
            pltpu.sync_copy(rows_v, out_hbm.at[pl.ds(off, chunk)])
            return carry

        lax.fori_loop(0, n_chunks, body, 0)

    return k(table, idx)


def _fuse_body(D, posd, offs, g_ref, pos_ref, meta_ref, md_ref, w_ref,
               gam_ref, bet_ref, out_ref):
    meta = meta_ref[...]
    tt = meta[:, 0:1]
    emb = g_ref[...][:, :D]
    emb = emb * (1.0 + jnp.where(tt == 3, jnp.tanh(md_ref[...]), 0.0))

    j = lax.broadcasted_iota(jnp.int32, (1, w_ref.shape[0]), 1)
    is_atom = tt == 1
    is_bond = tt == 2
    oh = j == (tt + offs[0])
    oh |= is_atom & ((j == meta[:, 1:2] + offs[1])
                     | (j == meta[:, 2:3] + offs[2])
                     | (j == meta[:, 3:4] + offs[3])
                     | (j == meta[:, 4:5] + offs[4]))
    oh |= is_bond & ((j == meta[:, 5:6] + offs[5])
                     | (j == meta[:, 6:7] + offs[6])
                     | (j == meta[:, 7:8] + offs[7]))
    aug = jnp.dot(oh.astype(jnp.float32), w_ref[...],
                  preferred_element_type=jnp.float32)

    T = emb.shape[0]
    hid = out_ref.shape[1]
    contrib = aug[:, :hid]
    ep = jnp.concatenate([emb, pos_ref[...]], axis=1)
    s = jnp.sum(ep, axis=1, keepdims=True) + aug[:, hid:hid + 1]
    ss = jnp.sum(ep * ep, axis=1, keepdims=True) + aug[:, hid + 1:hid + 2]
    mean = s * (1.0 / hid)
    var = ss * (1.0 / hid) - mean * mean
    inv = lax.rsqrt(var + EPS)
    base = jnp.concatenate(
        [ep, jnp.zeros((T, hid - D - posd), jnp.float32)], axis=1)
    full = base + contrib
    out_ref[...] = (full - mean) * inv * gam_ref[...] + bet_ref[...]


@functools.partial(jax.jit, static_argnames=("D", "posd", "offs", "block"))
def _tc_fuse(gathered, pos, meta, md, w, gamma, beta, D, posd, offs,
             block=512):
    N, Dp = gathered.shape
    hid = w.shape[1] - 2
    assert N % block == 0
    grid = (N // block,)

    def tok_spec(d):
        return pl.BlockSpec((block, d), lambda i: (i, 0))

    def full_spec(shape):
        return pl.BlockSpec(shape, lambda i: (0, 0))

    return pl.pallas_call(
        functools.partial(_fuse_body, D, posd, offs),
        grid=grid,
        in_specs=[
            tok_spec(Dp), tok_spec(posd), tok_spec(meta.shape[1]), tok_spec(1),
            full_spec(w.shape), full_spec((1, hid)), full_spec((1, hid)),
        ],
        out_specs=pl.BlockSpec((block, hid), lambda i: (i, 0)),
        out_shape=jax.ShapeDtypeStruct((N, hid), jnp.float32),
    )(gathered, pos, meta, md, w, gamma.reshape(1, hid), beta.reshape(1, hid))


def kernel(input_ids, token_type_ids, pos_embeds, pos_embeds_shape,
           atom_props, bond_props, mol_desc, emb_table, type_table,
           in_ring_table, charge_table, hybrid_table, chirality_table,
           aromatic_table, conjugated_table, stereo_table, ln_gamma, ln_beta):
    B, L = input_ids.shape
    N = B * L
    posd = pos_embeds.shape[1] // L
    D = emb_table.shape[1]
    hid = ln_gamma.shape[0]

    ids = input_ids.reshape(N).astype(jnp.int32)
    Dp = ((D + 127) // 128) * 128
    table_p = jnp.pad(emb_table, ((0, 0), (0, Dp - D)))
    gathered = _sc_gather(table_p, ids)

    meta = jnp.concatenate([
        token_type_ids.reshape(N, 1),
        atom_props.reshape(N, 4),
        bond_props.reshape(N, 3),
    ], axis=1).astype(jnp.int32)
    pos = pos_embeds.reshape(N, posd)
    md = mol_desc.reshape(N, 1)

    t0 = D + posd
    p0 = t0 + D
    per4 = in_ring_table.shape[1]
    per3 = aromatic_table.shape[1]
    tables = [
        (type_table, t0),
        (in_ring_table, p0),
        (charge_table, p0 + per4),
        (hybrid_table, p0 + 2 * per4),
        (chirality_table, p0 + 3 * per4),
        (aromatic_table, p0),
        (conjugated_table, p0 + per3),
        (stereo_table, p0 + 2 * per3),
    ]
    w = jnp.zeros((128, hid + 2), jnp.float32)
    offs = []
    r = 0
    for tab, col in tables:
        n, d = tab.shape
        w = w.at[r:r + n, col:col + d].set(tab)
        w = w.at[r:r + n, hid].set(jnp.sum(tab, axis=1))
        w = w.at[r:r + n, hid + 1].set(jnp.sum(tab * tab, axis=1))
        offs.append(r)
        r += n

    out = _tc_fuse(gathered, pos, meta, md, w, ln_gamma, ln_beta,
                   D, posd, tuple(offs))
    return out.reshape(B, L, hid)


# speedup vs baseline: 6.5568x; 1.0056x over previous
import functools

import jax
import jax.numpy as jnp
from jax import lax
from jax.experimental import pallas as pl
from jax.experimental.pallas import tpu as pltpu
from jax.experimental.pallas import tpu_sc as plsc

EPS = 1e-12


@functools.partial(jax.jit, static_argnames=("chunk",))
def _sc_gather(table, idx, chunk=128):
    V, D = table.shape
    N = idx.shape[0]
    info = plsc.get_sparse_core_info()
    NC, NS = info.num_cores, info.num_subcores
    NW = NC * NS
    assert N % (NW * chunk) == 0
    per_w = N // NW
    n_chunks = per_w // chunk
    mesh = plsc.VectorSubcoreMesh(core_axis_name="c", subcore_axis_name="s")

    @functools.partial(
        pl.kernel,
        mesh=mesh,
        out_type=jax.ShapeDtypeStruct((N, D), jnp.float32),
        scratch_types=[
            pltpu.VMEM((chunk,), jnp.int32),
            pltpu.VMEM((chunk, D), jnp.float32),
            pltpu.SemaphoreType.DMA,
        ],
    )
    def k(table_hbm, idx_hbm, out_hbm, idx_v, rows_v, sem):
        wid = lax.axis_index("s") * NC + lax.axis_index("c")
        base = wid * per_w

        def body(i, carry):
            off = base + i * chunk
            pltpu.sync_copy(idx_hbm.at[pl.ds(off, chunk)], idx_v)
            pltpu.async_copy(table_hbm.at[idx_v], rows_v, sem).wait()
            pltpu.sync_copy(rows_v, out_hbm.at[pl.ds(off, chunk)])
            return carry

        lax.fori_loop(0, n_chunks, body, 0)

    return k(table, idx)


def _fuse_body(D, posd, offs, g_ref, pos_ref, meta_ref, md_ref, w_ref,
               gam_ref, bet_ref, out_ref):
    meta = meta_ref[...]
    tt = meta[:, 0:1]
    emb = g_ref[...][:, :D]
    emb = emb * (1.0 + jnp.where(tt == 3, jnp.tanh(md_ref[...]), 0.0))

    j = lax.broadcasted_iota(jnp.int32, (1, w_ref.shape[0]), 1)
    is_atom = tt == 1
    is_bond = tt == 2
    zrow = w_ref.shape[0] - 1

    def sel(av, bv):
        return jnp.where(is_atom, av, jnp.where(is_bond, bv, zrow))

    k1 = tt + offs[0]
    k2 = sel(meta[:, 1:2] + offs[1], meta[:, 5:6] + offs[5])
    k3 = sel(meta[:, 2:3] + offs[2], meta[:, 6:7] + offs[6])
    k4 = sel(meta[:, 3:4] + offs[3], meta[:, 7:8] + offs[7])
    k5 = jnp.where(is_atom, meta[:, 4:5] + offs[4], zrow)
    oh = (j == k1) | (j == k2) | (j == k3) | (j == k4) | (j == k5)
    aug = jnp.dot(oh.astype(jnp.float32), w_ref[...],
                  preferred_element_type=jnp.float32)

    T = emb.shape[0]
    hid = out_ref.shape[1]
    contrib = aug[:, :hid]
    ep = jnp.concatenate([emb, pos_ref[...]], axis=1)
    s = jnp.sum(ep, axis=1, keepdims=True) + aug[:, hid:hid + 1]
    ss = jnp.sum(ep * ep, axis=1, keepdims=True) + aug[:, hid + 1:hid + 2]
    mean = s * (1.0 / hid)
    var = ss * (1.0 / hid) - mean * mean
    inv = lax.rsqrt(var + EPS)
    base = jnp.concatenate(
        [ep, jnp.zeros((T, hid - D - posd), jnp.float32)], axis=1)
    full = base + contrib
    out_ref[...] = (full - mean) * inv * gam_ref[...] + bet_ref[...]


@functools.partial(jax.jit,
                   static_argnames=("D", "posd", "offs", "base_blk",
                                    "total_n", "block"))
def _tc_fuse_chunk(gathered_c, pos, meta, md, w, gamma, beta, buf,
                   D, posd, offs, base_blk, total_n, block=512):
    Nc, Dp = gathered_c.shape
    hid = w.shape[1] - 2
    assert Nc % block == 0
    grid = (Nc // block,)

    def chunk_spec(d):
        return pl.BlockSpec((block, d), lambda i: (i, 0))

    def off_spec(d):
        return pl.BlockSpec((block, d), lambda i: (base_blk + i, 0))

    def full_spec(shape):
        return pl.BlockSpec(shape, lambda i: (0, 0))

    in_specs = [
        chunk_spec(Dp), off_spec(posd), off_spec(meta.shape[1]), off_spec(1),
        full_spec(w.shape), full_spec((1, hid)), full_spec((1, hid)),
    ]
    args = [gathered_c, pos, meta, md, w,
            gamma.reshape(1, hid), beta.reshape(1, hid)]
    body = functools.partial(_fuse_body, D, posd, offs)
    extra = {}
    if buf is not None:
        in_specs.append(pl.BlockSpec(memory_space=pl.ANY))
        args.append(buf)
        extra["input_output_aliases"] = {7: 0}
        inner = body

        def body(*refs):
            return inner(*refs[:7], refs[8])

    return pl.pallas_call(
        body,
        grid=grid,
        in_specs=in_specs,
        out_specs=pl.BlockSpec((block, hid), lambda i: (base_blk + i, 0)),
        out_shape=jax.ShapeDtypeStruct((total_n, hid), jnp.float32),
        **extra,
    )(*args)


def kernel(input_ids, token_type_ids, pos_embeds, pos_embeds_shape,
           atom_props, bond_props, mol_desc, emb_table, type_table,
           in_ring_table, charge_table, hybrid_table, chirality_table,
           aromatic_table, conjugated_table, stereo_table, ln_gamma, ln_beta):
    B, L = input_ids.shape
    N = B * L
    posd = pos_embeds.shape[1] // L
    D = emb_table.shape[1]
    hid = ln_gamma.shape[0]

    ids = input_ids.reshape(N).astype(jnp.int32)
    Dp = ((D + 127) // 128) * 128
    table_p = jnp.pad(emb_table, ((0, 0), (0, Dp - D)))
    half = N // 2
    g0 = _sc_gather(table_p, ids[:half])
    g1 = _sc_gather(table_p, ids[half:])

    meta = jnp.concatenate([
        token_type_ids.reshape(N, 1),
        atom_props.reshape(N, 4),
        bond_props.reshape(N, 3),
    ], axis=1).astype(jnp.int32)
    pos = pos_embeds.reshape(N, posd)
    md = mol_desc.reshape(N, 1)

    t0 = D + posd
    p0 = t0 + D
    per4 = in_ring_table.shape[1]
    per3 = aromatic_table.shape[1]
    tables = [
        (type_table, t0),
        (in_ring_table, p0),
        (charge_table, p0 + per4),
        (hybrid_table, p0 + 2 * per4),
        (chirality_table, p0 + 3 * per4),
        (aromatic_table, p0),
        (conjugated_table, p0 + per3),
        (stereo_table, p0 + 2 * per3),
    ]
    w = jnp.zeros((128, hid + 2), jnp.float32)
    offs = []
    r = 0
    for tab, col in tables:
        n, d = tab.shape
        w = w.at[r:r + n, col:col + d].set(tab)
        w = w.at[r:r + n, hid].set(jnp.sum(tab, axis=1))
        w = w.at[r:r + n, hid + 1].set(jnp.sum(tab * tab, axis=1))
        offs.append(r)
        r += n

    block = 512
    buf = _tc_fuse_chunk(g0, pos, meta, md, w, ln_gamma, ln_beta, None,
                         D, posd, tuple(offs), 0, N, block)
    out = _tc_fuse_chunk(g1, pos, meta, md, w, ln_gamma, ln_beta, buf,
                         D, posd, tuple(offs), half // block, N, block)
    return out.reshape(B, L, hid)


# speedup vs baseline: 9.8310x; 1.4994x over previous
import functools

import jax
import jax.numpy as jnp
from jax import lax
from jax.experimental import pallas as pl
from jax.experimental.pallas import tpu as pltpu
from jax.experimental.pallas import tpu_sc as plsc

EPS = 1e-12


@functools.partial(jax.jit, static_argnames=("chunk",))
def _sc_gather(table, idx, chunk=128):
    V, D = table.shape
    N = idx.shape[0]
    info = plsc.get_sparse_core_info()
    NC, NS = info.num_cores, info.num_subcores
    NW = NC * NS
    assert N % (NW * chunk) == 0
    per_w = N // NW
    n_chunks = per_w // chunk
    mesh = plsc.VectorSubcoreMesh(core_axis_name="c", subcore_axis_name="s")

    @functools.partial(
        pl.kernel,
        mesh=mesh,
        out_type=jax.ShapeDtypeStruct((N, D), jnp.float32),
        scratch_types=[
            pltpu.VMEM((chunk,), jnp.int32),
            pltpu.VMEM((chunk, D), jnp.float32),
            pltpu.SemaphoreType.DMA,
        ],
    )
    def k(table_hbm, idx_hbm, out_hbm, idx_v, rows_v, sem):
        wid = lax.axis_index("s") * NC + lax.axis_index("c")
        base = wid * per_w

        def body(i, carry):
            off = base + i * chunk
            pltpu.sync_copy(idx_hbm.at[pl.ds(off, chunk)], idx_v)
            pltpu.async_copy(table_hbm.at[idx_v], rows_v, sem).wait()
            pltpu.sync_copy(rows_v, out_hbm.at[pl.ds(off, chunk)])
            return carry

        lax.fori_loop(0, n_chunks, body, 0)

    return k(table, idx)


def _fuse_body(D, posd, offs, g_ref, pos_ref, meta_ref, md_ref, w_ref,
               gam_ref, bet_ref, out_ref):
    tt = meta_ref[0:1, :]
    is_atom = tt == 1
    is_bond = tt == 2
    zrow = w_ref.shape[0] - 1
    jcol = lax.broadcasted_iota(jnp.int32, (w_ref.shape[0], 1), 0)

    def row(r, o):
        return meta_ref[r:r + 1, :] + o

    def sel(av, bv):
        return jnp.where(is_atom, av, jnp.where(is_bond, bv, zrow))

    k1 = tt + offs[0]
    k2 = sel(row(1, offs[1]), row(5, offs[5]))
    k3 = sel(row(2, offs[2]), row(6, offs[6]))
    k4 = sel(row(3, offs[3]), row(7, offs[7]))
    k5 = jnp.where(is_atom, row(4, offs[4]), zrow)
    ohT = ((jcol == k1) | (jcol == k2) | (jcol == k3) | (jcol == k4)
           | (jcol == k5))
    aug = lax.dot_general(ohT.astype(jnp.float32), w_ref[...],
                          (((0,), (0,)), ((), ())),
                          preferred_element_type=jnp.float32)

    hid = out_ref.shape[1]
    scale = 1.0 + jnp.where(tt == 3, jnp.tanh(md_ref[...]), 0.0)
    emb = g_ref[...][:, :D] * scale.T
    pos = pos_ref[...].T
    ep = jnp.concatenate([emb, pos], axis=1)
    s = jnp.sum(ep, axis=1, keepdims=True) + aug[:, hid:hid + 1]
    ss = jnp.sum(ep * ep, axis=1, keepdims=True) + aug[:, hid + 1:hid + 2]
    mean = s * (1.0 / hid)
    var = ss * (1.0 / hid) - mean * mean
    inv = lax.rsqrt(var + EPS)
    lo = D + posd
    out_ref[:, :lo] = (ep - mean) * inv * gam_ref[:, :lo] + bet_ref[:, :lo]
    out_ref[:, lo:] = ((aug[:, lo:hid] - mean) * inv * gam_ref[:, lo:]
                       + bet_ref[:, lo:])


@functools.partial(jax.jit,
                   static_argnames=("D", "posd", "offs", "base_blk",
                                    "total_n", "block"))
def _tc_fuse_chunk(gathered_c, pos, meta, md, w, gamma, beta, buf,
                   D, posd, offs, base_blk, total_n, block=512):
    Nc, Dp = gathered_c.shape
    hid = w.shape[1] - 2
    assert Nc % block == 0
    grid = (Nc // block,)

    def chunk_spec(d):
        return pl.BlockSpec((block, d), lambda i: (i, 0))

    def t_spec(k):
        return pl.BlockSpec((k, block), lambda i: (0, base_blk + i))

    def full_spec(shape):
        return pl.BlockSpec(shape, lambda i: (0, 0))

    in_specs = [
        chunk_spec(Dp), t_spec(posd), t_spec(meta.shape[0]), t_spec(1),
        full_spec(w.shape), full_spec((1, hid)), full_spec((1, hid)),
    ]
    args = [gathered_c, pos, meta, md, w,
            gamma.reshape(1, hid), beta.reshape(1, hid)]
    body = functools.partial(_fuse_body, D, posd, offs)
    extra = {}
    if buf is not None:
        in_specs.append(pl.BlockSpec(memory_space=pl.ANY))
        args.append(buf)
        extra["input_output_aliases"] = {7: 0}
        inner = body

        def body(*refs):
            return inner(*refs[:7], refs[8])

    return pl.pallas_call(
        body,
        grid=grid,
        in_specs=in_specs,
        out_specs=pl.BlockSpec((block, hid), lambda i: (base_blk + i, 0)),
        out_shape=jax.ShapeDtypeStruct((total_n, hid), jnp.float32),
        **extra,
    )(*args)


def kernel(input_ids, token_type_ids, pos_embeds, pos_embeds_shape,
           atom_props, bond_props, mol_desc, emb_table, type_table,
           in_ring_table, charge_table, hybrid_table, chirality_table,
           aromatic_table, conjugated_table, stereo_table, ln_gamma, ln_beta):
    B, L = input_ids.shape
    N = B * L
    posd = pos_embeds.shape[1] // L
    D = emb_table.shape[1]
    hid = ln_gamma.shape[0]

    ids = input_ids.reshape(N).astype(jnp.int32)
    Dp = ((D + 127) // 128) * 128
    table_p = jnp.pad(emb_table, ((0, 0), (0, Dp - D)))
    half = N // 2
    g0 = _sc_gather(table_p, ids[:half])
    g1 = _sc_gather(table_p, ids[half:])

    meta = jnp.concatenate([
        token_type_ids.reshape(1, N),
        atom_props.reshape(N, 4).T,
        bond_props.reshape(N, 3).T,
    ], axis=0).astype(jnp.int32)
    pos = pos_embeds.reshape(B, L, posd).transpose(2, 0, 1).reshape(posd, N)
    md = mol_desc.reshape(1, N)

    t0 = D + posd
    p0 = t0 + D
    per4 = in_ring_table.shape[1]
    per3 = aromatic_table.shape[1]
    tables = [
        (type_table, t0),
        (in_ring_table, p0),
        (charge_table, p0 + per4),
        (hybrid_table, p0 + 2 * per4),
        (chirality_table, p0 + 3 * per4),
        (aromatic_table, p0),
        (conjugated_table, p0 + per3),
        (stereo_table, p0 + 2 * per3),
    ]
    wrows = []
    offs = []
    r = 0
    for tab, col in tables:
        n, d = tab.shape
        wrows.append(jnp.concatenate([
            jnp.zeros((n, col), jnp.float32), tab,
            jnp.zeros((n, hid - col - d), jnp.float32),
            jnp.sum(tab, axis=1, keepdims=True),
            jnp.sum(tab * tab, axis=1, keepdims=True),
        ], axis=1))
        offs.append(r)
        r += n
    w = jnp.concatenate(
        wrows + [jnp.zeros((128 - r, hid + 2), jnp.float32)], axis=0)

    block = 512
    buf = _tc_fuse_chunk(g0, pos, meta, md, w, ln_gamma, ln_beta, None,
                         D, posd, tuple(offs), 0, N, block)
    out = _tc_fuse_chunk(g1, pos, meta, md, w, ln_gamma, ln_beta, buf,
                         D, posd, tuple(offs), half // block, N, block)
    return out.reshape(B, L, hid)


# speedup vs baseline: 11.2305x; 1.1424x over previous
import functools

import jax
import jax.numpy as jnp
from jax import lax
from jax.experimental import pallas as pl
from jax.experimental.pallas import tpu as pltpu
from jax.experimental.pallas import tpu_sc as plsc

EPS = 1e-12


@functools.partial(jax.jit, static_argnames=("chunk",))
def _sc_gather(table, idx, chunk=128):
    V, D = table.shape
    N = idx.shape[0]
    info = plsc.get_sparse_core_info()
    NC, NS = info.num_cores, info.num_subcores
    NW = NC * NS
    assert N % (NW * chunk) == 0
    per_w = N // NW
    n_chunks = per_w // chunk
    mesh = plsc.VectorSubcoreMesh(core_axis_name="c", subcore_axis_name="s")

    @functools.partial(
        pl.kernel,
        mesh=mesh,
        out_type=jax.ShapeDtypeStruct((N, D), jnp.float32),
        scratch_types=[
            pltpu.VMEM((chunk,), jnp.int32),
            pltpu.VMEM((chunk, D), jnp.float32),
            pltpu.SemaphoreType.DMA,
        ],
    )
    def k(table_hbm, idx_hbm, out_hbm, idx_v, rows_v, sem):
        wid = lax.axis_index("s") * NC + lax.axis_index("c")
        base = wid * per_w

        def body(i, carry):
            off = base + i * chunk
            pltpu.sync_copy(idx_hbm.at[pl.ds(off, chunk)], idx_v)
            pltpu.async_copy(table_hbm.at[idx_v], rows_v, sem).wait()
            pltpu.sync_copy(rows_v, out_hbm.at[pl.ds(off, chunk)])
            return carry

        lax.fori_loop(0, n_chunks, body, 0)

    return k(table, idx)


def _fuse_body(D, posd, offs, g_ref, pos_ref, meta_ref, md_ref, w_ref,
               gam_ref, bet_ref, out_ref):
    tt = meta_ref[0:1, :]
    is_atom = tt == 1
    is_bond = tt == 2
    zrow = w_ref.shape[0] - 1
    jcol = lax.broadcasted_iota(jnp.int32, (w_ref.shape[0], 1), 0)

    def row(r, o):
        return meta_ref[r:r + 1, :] + o

    def sel(av, bv):
        return jnp.where(is_atom, av, jnp.where(is_bond, bv, zrow))

    k1 = tt + offs[0]
    k2 = sel(row(1, offs[1]), row(5, offs[5]))
    k3 = sel(row(2, offs[2]), row(6, offs[6]))
    k4 = sel(row(3, offs[3]), row(7, offs[7]))
    k5 = jnp.where(is_atom, row(4, offs[4]), zrow)
    ohT = ((jcol == k1) | (jcol == k2) | (jcol == k3) | (jcol == k4)
           | (jcol == k5))
    aug = lax.dot_general(ohT.astype(jnp.float32), w_ref[...],
                          (((0,), (0,)), ((), ())),
                          preferred_element_type=jnp.float32)

    hid = out_ref.shape[1]
    scale = 1.0 + jnp.where(tt == 3, jnp.tanh(md_ref[...]), 0.0)
    emb = g_ref[...][:, :D] * scale.T
    pos = pos_ref[...].T
    ep = jnp.concatenate([emb, pos], axis=1)
    s = jnp.sum(ep, axis=1, keepdims=True) + aug[:, hid:hid + 1]
    ss = jnp.sum(ep * ep, axis=1, keepdims=True) + aug[:, hid + 1:hid + 2]
    mean = s * (1.0 / hid)
    var = ss * (1.0 / hid) - mean * mean
    inv = lax.rsqrt(var + EPS)
    lo = D + posd
    out_ref[:, :lo] = (ep - mean) * inv * gam_ref[:, :lo] + bet_ref[:, :lo]
    out_ref[:, lo:] = ((aug[:, lo:hid] - mean) * inv * gam_ref[:, lo:]
                       + bet_ref[:, lo:])


@functools.partial(jax.jit,
                   static_argnames=("D", "posd", "offs", "base_blk",
                                    "total_n", "block"))
def _tc_fuse_chunk(gathered_c, pos, meta, md, w, gamma, beta, buf,
                   D, posd, offs, base_blk, total_n, block=512):
    Nc, Dp = gathered_c.shape
    hid = w.shape[1] - 2
    assert Nc % block == 0
    grid = (Nc // block,)

    def chunk_spec(d):
        return pl.BlockSpec((block, d), lambda i: (i, 0))

    def t_spec(k):
        return pl.BlockSpec((k, block), lambda i: (0, base_blk + i))

    def full_spec(shape):
        return pl.BlockSpec(shape, lambda i: (0, 0))

    in_specs = [
        chunk_spec(Dp), t_spec(posd), t_spec(meta.shape[0]), t_spec(1),
        full_spec(w.shape), full_spec((1, hid)), full_spec((1, hid)),
    ]
    args = [gathered_c, pos, meta, md, w,
            gamma.reshape(1, hid), beta.reshape(1, hid)]
    body = functools.partial(_fuse_body, D, posd, offs)
    extra = {}
    if buf is not None:
        in_specs.append(pl.BlockSpec(memory_space=pl.ANY))
        args.append(buf)
        extra["input_output_aliases"] = {7: 0}
        inner = body

        def body(*refs):
            return inner(*refs[:7], refs[8])

    return pl.pallas_call(
        body,
        grid=grid,
        in_specs=in_specs,
        out_specs=pl.BlockSpec((block, hid), lambda i: (base_blk + i, 0)),
        out_shape=jax.ShapeDtypeStruct((total_n, hid), jnp.float32),
        **extra,
    )(*args)


def kernel(input_ids, token_type_ids, pos_embeds, pos_embeds_shape,
           atom_props, bond_props, mol_desc, emb_table, type_table,
           in_ring_table, charge_table, hybrid_table, chirality_table,
           aromatic_table, conjugated_table, stereo_table, ln_gamma, ln_beta):
    B, L = input_ids.shape
    N = B * L
    posd = pos_embeds.shape[1] // L
    D = emb_table.shape[1]
    hid = ln_gamma.shape[0]

    ids = input_ids.reshape(N).astype(jnp.int32)
    Dp = ((D + 127) // 128) * 128
    table_p = jnp.pad(emb_table, ((0, 0), (0, Dp - D)))
    K = 4
    C = N // K
    gs = [_sc_gather(table_p, ids[c * C:(c + 1) * C], chunk=80)
          for c in range(K)]

    meta = jnp.concatenate([
        token_type_ids.reshape(1, N),
        atom_props.reshape(N, 4).T,
        bond_props.reshape(N, 3).T,
    ], axis=0).astype(jnp.int32)
    pos = pos_embeds.reshape(B, L, posd).transpose(2, 0, 1).reshape(posd, N)
    md = mol_desc.reshape(1, N)

    t0 = D + posd
    p0 = t0 + D
    per4 = in_ring_table.shape[1]
    per3 = aromatic_table.shape[1]
    tables = [
        (type_table, t0),
        (in_ring_table, p0),
        (charge_table, p0 + per4),
        (hybrid_table, p0 + 2 * per4),
        (chirality_table, p0 + 3 * per4),
        (aromatic_table, p0),
        (conjugated_table, p0 + per3),
        (stereo_table, p0 + 2 * per3),
    ]
    wrows = []
    offs = []
    r = 0
    for tab, col in tables:
        n, d = tab.shape
        wrows.append(jnp.concatenate([
            jnp.zeros((n, col), jnp.float32), tab,
            jnp.zeros((n, hid - col - d), jnp.float32),
            jnp.sum(tab, axis=1, keepdims=True),
            jnp.sum(tab * tab, axis=1, keepdims=True),
        ], axis=1))
        offs.append(r)
        r += n
    w = jnp.concatenate(
        wrows + [jnp.zeros((128 - r, hid + 2), jnp.float32)], axis=0)

    block = 1024
    buf = None
    for c in range(K):
        buf = _tc_fuse_chunk(gs[c], pos, meta, md, w, ln_gamma, ln_beta, buf,
                             D, posd, tuple(offs), c * (C // block), N, block)
    return buf.reshape(B, L, hid)
